# baseline (speedup 1.0000x reference)
import jax
import jax.numpy as jnp
from jax import lax
from jax.experimental import pallas as pl
from jax.experimental.pallas import tpu as pltpu

_BR = 512


def _bdot(a, b, dims=((1,), (0,))):
    return lax.dot_general(a.astype(jnp.bfloat16), b.astype(jnp.bfloat16),
                           (dims, ((), ())),
                           preferred_element_type=jnp.float32)


def _colsum_body(A_ref, d_ref, acc):
    r = pl.program_id(0)
    BR, N = A_ref.shape

    @pl.when(r == 0)
    def _():
        acc[...] = jnp.zeros_like(acc)

    blk = A_ref[...]
    a = acc[...]
    col_ids = lax.broadcasted_iota(jnp.int32, (8, N), 1)
    base_ids = lax.broadcasted_iota(jnp.int32, (8, N), 0) + r * BR
    for i in range(BR // 8):
        eye = jnp.where(base_ids + 8 * i == col_ids, 1.0, 0.0)
        a = a + (blk[8 * i:8 * i + 8] + eye)
    acc[...] = a

    @pl.when(r == pl.num_programs(0) - 1)
    def _():
        t = a[0:4] + a[4:8]
        t2 = t[0:2] + t[2:4]
        cs = t2[0:1] + t2[1:2]
        d_ref[...] = (cs + 1e-5) ** -0.5


def _lap_block(A_blk, dcol_blk, drow, r):
    BR, N = A_blk.shape
    row_ids = lax.broadcasted_iota(jnp.int32, (BR, N), 0) + r * BR
    col_ids = lax.broadcasted_iota(jnp.int32, (BR, N), 1)
    A_hat = A_blk + jnp.where(row_ids == col_ids, 1.0, 0.0)
    return ((dcol_blk * A_hat) * drow).astype(jnp.bfloat16)


def _bn_relu(h, g, b):
    mu = jnp.mean(h, axis=0, keepdims=True)
    xc = h - mu
    var = jnp.mean(xc * xc, axis=0, keepdims=True)
    return jnp.maximum(g * (xc / jnp.sqrt(var + 1e-5)) + b, 0.0)


def _layer1_body(A_ref, dcol_ref, drow_ref, x_ref, W0_ref, g0_ref, b0_ref,
                 Lb_ref, X1_ref, Hpre, xb):
    r = pl.program_id(0)
    BR = A_ref.shape[0]

    @pl.when(r == 0)
    def _():
        xb[...] = x_ref[...].astype(jnp.bfloat16)

    Lb = _lap_block(A_ref[...], dcol_ref[...], drow_ref[...], r)
    Lb_ref[...] = Lb
    Hpre[pl.ds(r * BR, BR), :] = lax.dot_general(
        Lb, xb[...], ((((1,), (0,))), ((), ())),
        preferred_element_type=jnp.float32)

    @pl.when(r == pl.num_programs(0) - 1)
    def _():
        h = _bdot(Hpre[...], W0_ref[...])
        X1_ref[...] = _bn_relu(h, g0_ref[...], b0_ref[...])


def _layers23_body(Lb_ref, X1_ref, W1_ref, g1_ref, b1_ref, W2_ref, g2_ref,
                   b2_ref, decW_ref, decb_ref, seqW_ref, seqb_ref,
                   hdec_ref, seq_ref, Hpre, X2, Yb):
    p = pl.program_id(0)
    r = pl.program_id(1)
    BR = Lb_ref.shape[0]
    F = W1_ref.shape[0]

    @pl.when((r == 0) & (p == 0))
    def _():
        Yb[...] = X1_ref[...].astype(jnp.bfloat16)

    @pl.when((r == 0) & (p == 1))
    def _():
        Yb[...] = X2[...].astype(jnp.bfloat16)

    Hpre[pl.ds(r * BR, BR), :] = lax.dot_general(
        Lb_ref[...], Yb[...], ((((1,), (0,))), ((), ())),
        preferred_element_type=jnp.float32)

    @pl.when((r == pl.num_programs(1) - 1) & (p == 0))
    def _():
        h = _bdot(Hpre[...], W1_ref[...])
        X2[...] = _bn_relu(h, g1_ref[...], b1_ref[...])

    @pl.when((r == pl.num_programs(1) - 1) & (p == 1))
    def _():
        h = _bdot(Hpre[...], W2_ref[...])
        x3 = _bn_relu(h, g2_ref[...], b2_ref[...])
        x1 = X1_ref[...]
        x2 = X2[...]
        dW = decW_ref[...]
        hdec_ref[...] = (_bdot(x1, dW[0:F]) + _bdot(x2, dW[F:2 * F])
                         + _bdot(x3, dW[2 * F:3 * F]) + decb_ref[...])
        sW = seqW_ref[...]
        s = (_bdot(x1, sW[0:F]) + _bdot(x2, sW[F:2 * F])
             + _bdot(x3, sW[2 * F:3 * F]) + seqb_ref[...])
        m = jnp.max(s, axis=-1, keepdims=True)
        lse = jnp.log(jnp.sum(jnp.exp(s - m), axis=-1, keepdims=True))
        seq_ref[...] = s - m - lse


def _decoder_body(hdec_ref, out_ref):
    i = pl.program_id(0)
    BM = out_ref.shape[0]
    hb = hdec_ref[pl.ds(i * BM, BM), :]
    logits = _bdot(hb, hdec_ref[...], ((1,), (1,)))
    out_ref[...] = jax.nn.sigmoid(logits)


def kernel(adj, x, W0, g0, beta0, W1, g1, beta1, W2, g2, beta2,
           dec_W, dec_b, seq_W, seq_b):
    Bb, N, _ = adj.shape
    A = adj.reshape(N, N)
    xf = x.reshape(N, x.shape[-1])
    F = W0.shape[1]
    S = seq_W.shape[1]
    g0r, b0r = g0.reshape(1, F), beta0.reshape(1, F)
    g1r, b1r = g1.reshape(1, F), beta1.reshape(1, F)
    g2r, b2r = g2.reshape(1, F), beta2.reshape(1, F)
    decbr = dec_b.reshape(1, -1)
    seqbr = seq_b.reshape(1, -1)
    R = N // _BR
    f32 = jnp.float32
    cparams = lambda nd: pltpu.CompilerParams(
        dimension_semantics=("arbitrary",) * nd,
        vmem_limit_bytes=100 * 1024 * 1024)

    drow = pl.pallas_call(
        _colsum_body,
        grid=(R,),
        in_specs=[pl.BlockSpec((_BR, N), lambda r: (r, 0))],
        out_specs=pl.BlockSpec((1, N), lambda r: (0, 0)),
        out_shape=jax.ShapeDtypeStruct((1, N), f32),
        scratch_shapes=[pltpu.VMEM((8, N), f32)],
        compiler_params=cparams(1),
    )(A)
    dcol = drow.reshape(N, 1)

    Lb16, X1 = pl.pallas_call(
        _layer1_body,
        grid=(R,),
        in_specs=[
            pl.BlockSpec((_BR, N), lambda r: (r, 0)),
            pl.BlockSpec((_BR, 1), lambda r: (r, 0)),
            pl.BlockSpec((1, N), lambda r: (0, 0)),
            pl.BlockSpec((N, xf.shape[1]), lambda r: (0, 0)),
            pl.BlockSpec(W0.shape, lambda r: (0, 0)),
            pl.BlockSpec((1, F), lambda r: (0, 0)),
            pl.BlockSpec((1, F), lambda r: (0, 0)),
        ],
        out_specs=[
            pl.BlockSpec((_BR, N), lambda r: (r, 0)),
            pl.BlockSpec((N, F), lambda r: (0, 0)),
        ],
        out_shape=[
            jax.ShapeDtypeStruct((N, N), jnp.bfloat16),
            jax.ShapeDtypeStruct((N, F), f32),
        ],
        scratch_shapes=[
            pltpu.VMEM((N, xf.shape[1]), f32),
            pltpu.VMEM((N, xf.shape[1]), jnp.bfloat16),
        ],
        compiler_params=cparams(1),
    )(A, dcol, drow, xf, W0, g0r, b0r)

    hdec, seq = pl.pallas_call(
        _layers23_body,
        grid=(2, R),
        in_specs=[
            pl.BlockSpec((_BR, N), lambda p, r: (r, 0)),
            pl.BlockSpec((N, F), lambda p, r: (0, 0)),
            pl.BlockSpec(W1.shape, lambda p, r: (0, 0)),
            pl.BlockSpec((1, F), lambda p, r: (0, 0)),
            pl.BlockSpec((1, F), lambda p, r: (0, 0)),
            pl.BlockSpec(W2.shape, lambda p, r: (0, 0)),
            pl.BlockSpec((1, F), lambda p, r: (0, 0)),
            pl.BlockSpec((1, F), lambda p, r: (0, 0)),
            pl.BlockSpec(dec_W.shape, lambda p, r: (0, 0)),
            pl.BlockSpec((1, dec_W.shape[1]), lambda p, r: (0, 0)),
            pl.BlockSpec(seq_W.shape, lambda p, r: (0, 0)),
            pl.BlockSpec((1, S), lambda p, r: (0, 0)),
        ],
        out_specs=[
            pl.BlockSpec((N, dec_W.shape[1]), lambda p, r: (0, 0)),
            pl.BlockSpec((N, S), lambda p, r: (0, 0)),
        ],
        out_shape=[
            jax.ShapeDtypeStruct((N, dec_W.shape[1]), f32),
            jax.ShapeDtypeStruct((N, S), f32),
        ],
        scratch_shapes=[
            pltpu.VMEM((N, F), f32),
            pltpu.VMEM((N, F), f32),
            pltpu.VMEM((N, F), jnp.bfloat16),
        ],
        compiler_params=cparams(2),
    )(Lb16, X1, W1, g1r, b1r, W2, g2r, b2r, dec_W, decbr, seq_W, seqbr)

    cmap = pl.pallas_call(
        _decoder_body,
        grid=(R,),
        in_specs=[pl.BlockSpec((N, dec_W.shape[1]), lambda i: (0, 0))],
        out_specs=pl.BlockSpec((_BR, N), lambda i: (i, 0)),
        out_shape=jax.ShapeDtypeStruct((N, N), f32),
        compiler_params=cparams(1),
    )(hdec)

    return (cmap.reshape(Bb, N, N), seq.reshape(Bb, N, S))
